# baseline (speedup 1.0000x reference)
import functools

import jax
import jax.numpy as jnp
import numpy as np
from jax import lax
from jax.experimental import pallas as pl
from jax.experimental.pallas import tpu as pltpu
from jax.experimental.pallas import tpu_sc as plsc

N = 10000
NP = 10240
E = 320000
NFEAT = 128
NHID = 128
NCLASS = 40
D2P = 64

NC = 2
NS = 16
NW = NC * NS
CH = 64
NBG = 4
NBS = 2
GB = 16
KJ = 160
TOTCH = NW * KJ
EP = TOTCH * CH
RPT = NP // NS
ZR = CH


def _unpack_perm(D):
    perm = np.empty((D,), np.int64)
    for g in range(D // 32):
        for i in range(16):
            perm[32 * g + i] = 32 * g + 2 * i
            perm[32 * g + 16 + i] = 32 * g + 2 * i + 1
    inv = np.empty_like(perm)
    inv[perm] = np.arange(D)
    return inv


def _make_sc_agg(D, k0):
    k1 = TOTCH // NS - k0
    mesh = plsc.VectorSubcoreMesh(core_axis_name="c", subcore_axis_name="s")

    @functools.partial(
        pl.kernel,
        out_type=jax.ShapeDtypeStruct((NC, NP, D), jnp.float32),
        mesh=mesh,
        compiler_params=pltpu.CompilerParams(use_tc_tiling_on_sc=False),
        scratch_types=[
            pltpu.VMEM((GB, CH), jnp.int32),
            pltpu.VMEM((GB, CH), jnp.int32),
            pltpu.VMEM((GB, CH), jnp.float32),
            [pltpu.VMEM((CH, D // 2), jnp.int32)] * NBG,
            [pltpu.VMEM((CH, D), jnp.float32)] * NBS,
            pltpu.VMEM_SHARED((NP, D), jnp.float32),
            [pltpu.SemaphoreType.DMA] * NBG,
            [pltpu.SemaphoreType.DMA] * NBS,
        ],
    )
    def agg(support, srcm, tgtm, mvals, out, src_v, tgt_v, mv_v, rowsb, rowsf,
            acc, sg, ss):
        cid = lax.axis_index("c")
        sid = lax.axis_index("s")
        kc = jnp.where(cid == 0, k0, k1)
        crow = jnp.where(cid == 0, sid * k0, NS * k0 + sid * k1)

        zeros16 = jnp.zeros((16,), jnp.float32)

        def zrow(r, carry):
            for f in range(D // 16):
                rowsf[0][r, pl.ds(f * 16, 16)] = zeros16
            return carry

        lax.fori_loop(0, ZR, zrow, 0)

        base = sid * RPT

        def zcp(i, carry):
            pltpu.sync_copy(rowsf[0], acc.at[pl.ds(base + i * ZR, ZR), :])
            return carry

        lax.fori_loop(0, RPT // ZR, zcp, 0)
        plsc.subcore_barrier()

        dn = lax.GatherDimensionNumbers(
            offset_dims=(), collapsed_slice_dims=(0,), start_index_map=(0,))

        def g_start(j, b):
            pltpu.async_copy(support.at[src_v.at[j]], rowsb[b], sg[b])

        def g_wait(j, b):
            pltpu.make_async_copy(support.at[src_v.at[j]], rowsb[b], sg[b]).wait()

        def s_start(j, b):
            pltpu.async_copy(rowsf[b], acc.at[tgt_v.at[j]], ss[b], add=True)

        def s_wait(j, b):
            pltpu.make_async_copy(rowsf[b], acc.at[tgt_v.at[j]], ss[b]).wait()

        himask = jnp.full((16,), -65536, jnp.int32)

        def scale(j, bg, bs):
            def sgrp(g, c2):
                mv16 = mv_v[j, pl.ds(g * 16, 16)]
                for i in range(16):
                    m = lax.gather(
                        mv16, jnp.full((16, 1), i, jnp.int32), dn, (1,),
                        mode=lax.GatherScatterMode.PROMISE_IN_BOUNDS)
                    e = g * 16 + i
                    for g2 in range(D // 32):
                        w = rowsb[bg][e, pl.ds(g2 * 16, 16)]
                        lo = lax.bitcast_convert_type(w << 16, jnp.float32)
                        hi = lax.bitcast_convert_type(w & himask, jnp.float32)
                        rowsf[bs][e, pl.ds(g2 * 32, 16)] = lo * m
                        rowsf[bs][e, pl.ds(g2 * 32 + 16, 16)] = hi * m
                return c2

            lax.fori_loop(0, CH // 16, sgrp, 0)

        def group(gr, carry):
            row = crow + gr * GB
            pltpu.sync_copy(srcm.at[pl.ds(row, GB), :], src_v)
            pltpu.sync_copy(tgtm.at[pl.ds(row, GB), :], tgt_v)
            pltpu.sync_copy(mvals.at[pl.ds(row, GB), :], mv_v)

            for b in range(NBG - 1):
                g_start(b, b)

            def quad(q, c1):
                for b in range(NBG):
                    j = q * NBG + b
                    bs = b % NBS
                    g_wait(j, b)

                    if b >= NBS:
                        s_wait(j - NBS, bs)
                    else:
                        @pl.when(q > 0)
                        def _():
                            s_wait(j - NBS, bs)

                    scale(j, b, bs)
                    s_start(j, bs)
                    bp = (b + NBG - 1) % NBG

                    @pl.when(j + NBG - 1 < GB)
                    def _():
                        g_start(j + NBG - 1, bp)
                return c1

            lax.fori_loop(0, GB // NBG, quad, 0)
            s_wait(GB - NBS, (GB - NBS) % NBS)
            s_wait(GB - 1, (GB - 1) % NBS)
            return carry

        lax.fori_loop(0, kc // GB, group, 0)
        plsc.subcore_barrier()

        def wb(i, carry):
            sl = pl.ds(base + i * ZR, ZR)
            pltpu.sync_copy(acc.at[sl, :], out.at[cid, sl, :])
            return carry

        lax.fori_loop(0, RPT // ZR, wb, 0)

    return agg


_sc_agg_128 = _make_sc_agg(NHID, 144)
_sc_agg_64 = _make_sc_agg(D2P, 144)


def _mm1_body(x_ref, w_ref, o_ref):
    r = jnp.dot(x_ref[...], w_ref[...], preferred_element_type=jnp.float32)
    o_ref[...] = r.astype(jnp.bfloat16)


def _layer2_body(p_ref, b1_ref, w2_ref, o_ref):
    h = jax.nn.relu(p_ref[0] + p_ref[1] + b1_ref[...])
    r = jnp.dot(h, w2_ref[...], preferred_element_type=jnp.float32)
    o_ref[...] = r.astype(jnp.bfloat16)


def _final_body(q_ref, b2_ref, o_ref):
    z = q_ref[0] + q_ref[1] + b2_ref[...]
    col = lax.broadcasted_iota(jnp.int32, (NP, D2P), 1)
    valid = col < NCLASS
    zm = jnp.where(valid, z, -jnp.inf)
    m = jnp.max(zm, axis=1, keepdims=True)
    s = jnp.sum(jnp.where(valid, jnp.exp(z - m), 0.0), axis=1, keepdims=True)
    o_ref[...] = z - m - jnp.log(s)


def _pack_words(bf):
    r, d = bf.shape
    return lax.bitcast_convert_type(bf.reshape(r, d // 2, 2), jnp.int32)


def kernel(x, src, tgt, Mtgt, W1, b1, W2, b2):
    pad = EP - E
    srcp = jnp.pad(src, (0, pad)).reshape(TOTCH, CH)
    tgtp = jnp.pad(tgt, (0, pad)).reshape(TOTCH, CH)
    mvp = jnp.pad(Mtgt, (0, pad)).reshape(TOTCH, CH)

    inv1 = _unpack_perm(NHID)
    inv2 = _unpack_perm(D2P)
    w1p = W1[:, inv1]
    w2p = jnp.pad(W2, ((0, 0), (0, D2P - NCLASS)))[:, inv2]
    b2p = jnp.pad(b2, (0, D2P - NCLASS))

    xp = jnp.pad(x, ((0, NP - N), (0, 0)))
    support1 = pl.pallas_call(
        _mm1_body,
        out_shape=jax.ShapeDtypeStruct((NP, NHID), jnp.bfloat16),
    )(xp, w1p)

    parts1 = _sc_agg_128(_pack_words(support1), srcp, tgtp, mvp)

    support2 = pl.pallas_call(
        _layer2_body,
        out_shape=jax.ShapeDtypeStruct((NP, D2P), jnp.bfloat16),
    )(parts1, b1, w2p)

    parts2 = _sc_agg_64(_pack_words(support2), srcp, tgtp, mvp)

    outp = pl.pallas_call(
        _final_body,
        out_shape=jax.ShapeDtypeStruct((NP, D2P), jnp.float32),
    )(parts2, b2p)

    return outp[:N, :NCLASS]


# speedup vs baseline: 1.1114x; 1.1114x over previous
import functools

import jax
import jax.numpy as jnp
import numpy as np
from jax import lax
from jax.experimental import pallas as pl
from jax.experimental.pallas import tpu as pltpu
from jax.experimental.pallas import tpu_sc as plsc

N = 10000
NP = 10240
E = 320000
NFEAT = 128
NHID = 128
NCLASS = 40
D2P = 64

NC = 2
NS = 16
NW = NC * NS
CH = 64
NBG = 4
NBS = 2
GB = 16
KJ = 160
TOTCH = NW * KJ
EP = TOTCH * CH
RPT = NP // NS
ZR = CH


def _unpack_perm(D):
    perm = np.empty((D,), np.int64)
    for g in range(D // 32):
        for i in range(16):
            perm[32 * g + i] = 32 * g + 2 * i
            perm[32 * g + 16 + i] = 32 * g + 2 * i + 1
    inv = np.empty_like(perm)
    inv[perm] = np.arange(D)
    return inv


def _make_sc_agg(D, k0):
    k1 = TOTCH // NS - k0
    mesh = plsc.VectorSubcoreMesh(core_axis_name="c", subcore_axis_name="s")

    @functools.partial(
        pl.kernel,
        out_type=jax.ShapeDtypeStruct((NC, NP, D), jnp.float32),
        mesh=mesh,
        compiler_params=pltpu.CompilerParams(use_tc_tiling_on_sc=False),
        scratch_types=[
            pltpu.VMEM((GB, CH), jnp.int32),
            pltpu.VMEM((GB, CH), jnp.int32),
            pltpu.VMEM((GB, CH), jnp.float32),
            [pltpu.VMEM((CH, D // 2), jnp.int32)] * NBG,
            [pltpu.VMEM((CH, D), jnp.float32)] * NBS,
            pltpu.VMEM_SHARED((NP, D), jnp.float32),
            [pltpu.SemaphoreType.DMA] * NBG,
            [pltpu.SemaphoreType.DMA] * NBS,
        ],
    )
    def agg(support, srcm, tgtm, mvals, out, src_v, tgt_v, mv_v, rowsb, rowsf,
            acc, sg, ss):
        cid = lax.axis_index("c")
        sid = lax.axis_index("s")
        kc = jnp.where(cid == 0, k0, k1)
        crow = jnp.where(cid == 0, sid * k0, NS * k0 + sid * k1)

        zeros16 = jnp.zeros((16,), jnp.float32)

        def zrow(r, carry):
            for f in range(D // 16):
                rowsf[0][r, pl.ds(f * 16, 16)] = zeros16
            return carry

        lax.fori_loop(0, ZR, zrow, 0)

        base = sid * RPT

        def zcp(i, carry):
            pltpu.sync_copy(rowsf[0], acc.at[pl.ds(base + i * ZR, ZR), :])
            return carry

        lax.fori_loop(0, RPT // ZR, zcp, 0)
        plsc.subcore_barrier()

        dn = lax.GatherDimensionNumbers(
            offset_dims=(), collapsed_slice_dims=(0,), start_index_map=(0,))

        def g_start(j, b):
            pltpu.async_copy(support.at[src_v.at[j]], rowsb[b], sg[b])

        def g_wait(j, b):
            pltpu.make_async_copy(support.at[src_v.at[j]], rowsb[b], sg[b]).wait()

        def s_start(j, b):
            pltpu.async_copy(rowsf[b], acc.at[tgt_v.at[j]], ss[b], add=True)

        def s_wait(j, b):
            pltpu.make_async_copy(rowsf[b], acc.at[tgt_v.at[j]], ss[b]).wait()

        himask = jnp.full((16,), -65536, jnp.int32)

        def scale(j, bg, bs):
            def sgrp(g, c2):
                mv16 = mv_v[j, pl.ds(g * 16, 16)]
                for i in range(16):
                    m = lax.gather(
                        mv16, jnp.full((16, 1), i, jnp.int32), dn, (1,),
                        mode=lax.GatherScatterMode.PROMISE_IN_BOUNDS)
                    e = g * 16 + i
                    for g2 in range(D // 32):
                        w = rowsb[bg][e, pl.ds(g2 * 16, 16)]
                        lo = lax.bitcast_convert_type(w << 16, jnp.float32)
                        hi = lax.bitcast_convert_type(w & himask, jnp.float32)
                        rowsf[bs][e, pl.ds(g2 * 32, 16)] = lo * m
                        rowsf[bs][e, pl.ds(g2 * 32 + 16, 16)] = hi * m
                return c2

            lax.fori_loop(0, CH // 16, sgrp, 0)

        def group(gr, carry):
            row = crow + gr * GB
            pltpu.sync_copy(srcm.at[pl.ds(row, GB), :], src_v)
            pltpu.sync_copy(tgtm.at[pl.ds(row, GB), :], tgt_v)
            pltpu.sync_copy(mvals.at[pl.ds(row, GB), :], mv_v)

            for b in range(NBG - 1):
                g_start(b, b)

            def quad(q, c1):
                for b in range(NBG):
                    j = q * NBG + b
                    bs = b % NBS
                    g_wait(j, b)

                    if b >= NBS:
                        s_wait(j - NBS, bs)
                    else:
                        @pl.when(q > 0)
                        def _():
                            s_wait(j - NBS, bs)

                    scale(j, b, bs)
                    s_start(j, bs)
                    bp = (b + NBG - 1) % NBG

                    @pl.when(j + NBG - 1 < GB)
                    def _():
                        g_start(j + NBG - 1, bp)
                return c1

            lax.fori_loop(0, GB // NBG, quad, 0)
            s_wait(GB - NBS, (GB - NBS) % NBS)
            s_wait(GB - 1, (GB - 1) % NBS)
            return carry

        lax.fori_loop(0, kc // GB, group, 0)
        plsc.subcore_barrier()

        def wb(i, carry):
            sl = pl.ds(base + i * ZR, ZR)
            pltpu.sync_copy(acc.at[sl, :], out.at[cid, sl, :])
            return carry

        lax.fori_loop(0, RPT // ZR, wb, 0)

    return agg


_sc_agg_128 = _make_sc_agg(NHID, 176)
_sc_agg_64 = _make_sc_agg(D2P, 192)


def _mm1_body(x_ref, w_ref, o_ref):
    r = jnp.dot(x_ref[...], w_ref[...], preferred_element_type=jnp.float32)
    o_ref[...] = r.astype(jnp.bfloat16)


def _layer2_body(p_ref, b1_ref, w2_ref, o_ref):
    h = jax.nn.relu(p_ref[0] + p_ref[1] + b1_ref[...])
    r = jnp.dot(h, w2_ref[...], preferred_element_type=jnp.float32)
    o_ref[...] = r.astype(jnp.bfloat16)


def _final_body(q_ref, b2_ref, o_ref):
    z = q_ref[0] + q_ref[1] + b2_ref[...]
    col = lax.broadcasted_iota(jnp.int32, (NP, D2P), 1)
    valid = col < NCLASS
    zm = jnp.where(valid, z, -jnp.inf)
    m = jnp.max(zm, axis=1, keepdims=True)
    s = jnp.sum(jnp.where(valid, jnp.exp(z - m), 0.0), axis=1, keepdims=True)
    o_ref[...] = z - m - jnp.log(s)


def _pack_words(bf):
    r, d = bf.shape
    return lax.bitcast_convert_type(bf.reshape(r, d // 2, 2), jnp.int32)


def kernel(x, src, tgt, Mtgt, W1, b1, W2, b2):
    pad = EP - E
    srcp = jnp.pad(src, (0, pad)).reshape(TOTCH, CH)
    tgtp = jnp.pad(tgt, (0, pad)).reshape(TOTCH, CH)
    mvp = jnp.pad(Mtgt, (0, pad)).reshape(TOTCH, CH)

    inv1 = _unpack_perm(NHID)
    inv2 = _unpack_perm(D2P)
    w1p = W1[:, inv1]
    w2p = jnp.pad(W2, ((0, 0), (0, D2P - NCLASS)))[:, inv2]
    b2p = jnp.pad(b2, (0, D2P - NCLASS))

    xp = jnp.pad(x, ((0, NP - N), (0, 0)))
    support1 = pl.pallas_call(
        _mm1_body,
        out_shape=jax.ShapeDtypeStruct((NP, NHID), jnp.bfloat16),
    )(xp, w1p)

    parts1 = _sc_agg_128(_pack_words(support1), srcp, tgtp, mvp)

    support2 = pl.pallas_call(
        _layer2_body,
        out_shape=jax.ShapeDtypeStruct((NP, D2P), jnp.bfloat16),
    )(parts1, b1, w2p)

    parts2 = _sc_agg_64(_pack_words(support2), srcp, tgtp, mvp)

    outp = pl.pallas_call(
        _final_body,
        out_shape=jax.ShapeDtypeStruct((NP, D2P), jnp.float32),
    )(parts2, b2p)

    return outp[:N, :NCLASS]


# speedup vs baseline: 1.2187x; 1.0965x over previous
import functools

import jax
import jax.numpy as jnp
import numpy as np
from jax import lax
from jax.experimental import pallas as pl
from jax.experimental.pallas import tpu as pltpu
from jax.experimental.pallas import tpu_sc as plsc

N = 10000
NP = 10240
E = 320000
NFEAT = 128
NHID = 128
NCLASS = 40
D2P = 64

NC = 2
NS = 16
NW = NC * NS
CH = 64
NBG = 4
NBS = 2
GB = 16
KJ = 160
TOTCH = NW * KJ
EP = TOTCH * CH
RPT = NP // NS
ZR = CH


def _unpack_perm(D):
    perm = np.empty((D,), np.int64)
    for g in range(D // 32):
        for i in range(16):
            perm[32 * g + i] = 16 * g + i
            perm[32 * g + 16 + i] = D // 2 + 16 * g + i
    inv = np.empty_like(perm)
    inv[perm] = np.arange(D)
    return inv


def _make_sc_agg(D, k0):
    k1 = TOTCH // NS - k0
    mesh = plsc.VectorSubcoreMesh(core_axis_name="c", subcore_axis_name="s")

    @functools.partial(
        pl.kernel,
        out_type=[jax.ShapeDtypeStruct((NP, D), jnp.float32)] * NC,
        mesh=mesh,
        compiler_params=pltpu.CompilerParams(use_tc_tiling_on_sc=False),
        scratch_types=[
            pltpu.VMEM((GB, CH), jnp.int32),
            pltpu.VMEM((GB, CH), jnp.int32),
            pltpu.VMEM((GB, CH), jnp.float32),
            [pltpu.VMEM((CH, D // 2), jnp.int32)] * NBG,
            [pltpu.VMEM((CH, D), jnp.float32)] * NBS,
            pltpu.VMEM_SHARED((NP, D), jnp.float32),
            [pltpu.SemaphoreType.DMA] * NBG,
            [pltpu.SemaphoreType.DMA] * NBS,
        ],
    )
    def agg(support, srcm, tgtm, mvals, out0, out1, src_v, tgt_v, mv_v, rowsb, rowsf,
            acc, sg, ss):
        cid = lax.axis_index("c")
        sid = lax.axis_index("s")
        kc = jnp.where(cid == 0, k0, k1)
        crow = jnp.where(cid == 0, sid * k0, NS * k0 + sid * k1)

        zeros16 = jnp.zeros((16,), jnp.float32)

        def zrow(r, carry):
            for f in range(D // 16):
                rowsf[0][r, pl.ds(f * 16, 16)] = zeros16
            return carry

        lax.fori_loop(0, ZR, zrow, 0)

        base = sid * RPT

        def zcp(i, carry):
            pltpu.sync_copy(rowsf[0], acc.at[pl.ds(base + i * ZR, ZR), :])
            return carry

        lax.fori_loop(0, RPT // ZR, zcp, 0)
        plsc.subcore_barrier()

        dn = lax.GatherDimensionNumbers(
            offset_dims=(), collapsed_slice_dims=(0,), start_index_map=(0,))

        def g_start(j, b):
            pltpu.async_copy(support.at[src_v.at[j]], rowsb[b], sg[b])

        def g_wait(j, b):
            pltpu.make_async_copy(support.at[src_v.at[j]], rowsb[b], sg[b]).wait()

        def s_start(j, b):
            pltpu.async_copy(rowsf[b], acc.at[tgt_v.at[j]], ss[b], add=True)

        def s_wait(j, b):
            pltpu.make_async_copy(rowsf[b], acc.at[tgt_v.at[j]], ss[b]).wait()

        himask = jnp.full((16,), -65536, jnp.int32)

        def scale(j, bg, bs):
            def sgrp(g, c2):
                mv16 = mv_v[j, pl.ds(g * 16, 16)]
                for i in range(16):
                    m = lax.gather(
                        mv16, jnp.full((16, 1), i, jnp.int32), dn, (1,),
                        mode=lax.GatherScatterMode.PROMISE_IN_BOUNDS)
                    e = g * 16 + i
                    for g2 in range(D // 32):
                        w = rowsb[bg][e, pl.ds(g2 * 16, 16)]
                        lo = lax.bitcast_convert_type(w << 16, jnp.float32)
                        hi = lax.bitcast_convert_type(w & himask, jnp.float32)
                        rowsf[bs][e, pl.ds(g2 * 32, 16)] = lo * m
                        rowsf[bs][e, pl.ds(g2 * 32 + 16, 16)] = hi * m
                return c2

            lax.fori_loop(0, CH // 16, sgrp, 0)

        def group(gr, carry):
            row = crow + gr * GB
            pltpu.sync_copy(srcm.at[pl.ds(row, GB), :], src_v)
            pltpu.sync_copy(tgtm.at[pl.ds(row, GB), :], tgt_v)
            pltpu.sync_copy(mvals.at[pl.ds(row, GB), :], mv_v)

            for b in range(NBG - 1):
                g_start(b, b)

            def quad(q, c1):
                for b in range(NBG):
                    j = q * NBG + b
                    bs = b % NBS
                    g_wait(j, b)

                    if b >= NBS:
                        s_wait(j - NBS, bs)
                    else:
                        @pl.when(q > 0)
                        def _():
                            s_wait(j - NBS, bs)

                    scale(j, b, bs)
                    s_start(j, bs)
                    bp = (b + NBG - 1) % NBG

                    @pl.when(j + NBG - 1 < GB)
                    def _():
                        g_start(j + NBG - 1, bp)
                return c1

            lax.fori_loop(0, GB // NBG, quad, 0)
            s_wait(GB - NBS, (GB - NBS) % NBS)
            s_wait(GB - 1, (GB - 1) % NBS)
            return carry

        lax.fori_loop(0, kc // GB, group, 0)
        plsc.subcore_barrier()

        def wb0(i, carry):
            sl = pl.ds(base + i * ZR, ZR)
            pltpu.sync_copy(acc.at[sl, :], out0.at[sl, :])
            return carry

        def wb1(i, carry):
            sl = pl.ds(base + i * ZR, ZR)
            pltpu.sync_copy(acc.at[sl, :], out1.at[sl, :])
            return carry

        @pl.when(cid == 0)
        def _():
            lax.fori_loop(0, RPT // ZR, wb0, 0)

        @pl.when(cid == 1)
        def _():
            lax.fori_loop(0, RPT // ZR, wb1, 0)

    return agg


_sc_agg_128 = _make_sc_agg(NHID, 176)
_sc_agg_64 = _make_sc_agg(D2P, 176)


def _pack_halves(r):
    d = r.shape[1]
    a = r[:, : d // 2].astype(jnp.bfloat16)
    b = r[:, d // 2:].astype(jnp.bfloat16)
    au = lax.bitcast_convert_type(a, jnp.uint16).astype(jnp.uint32)
    bu = lax.bitcast_convert_type(b, jnp.uint16).astype(jnp.uint32)
    return lax.bitcast_convert_type(au | (bu << 16), jnp.int32)


def _mm1_body(x_ref, w_ref, o_ref):
    r = jnp.dot(x_ref[...], w_ref[...], preferred_element_type=jnp.float32)
    o_ref[...] = _pack_halves(r)


def _layer2_body(p0_ref, p1_ref, b1_ref, w2_ref, o_ref):
    h = jax.nn.relu(p0_ref[...] + p1_ref[...] + b1_ref[...])
    r = jnp.dot(h, w2_ref[...], preferred_element_type=jnp.float32)
    o_ref[...] = _pack_halves(r)


def _final_body(q0_ref, q1_ref, b2_ref, o_ref):
    z = q0_ref[...] + q1_ref[...] + b2_ref[...]
    col = lax.broadcasted_iota(jnp.int32, (NP, D2P), 1)
    valid = col < NCLASS
    zm = jnp.where(valid, z, -jnp.inf)
    m = jnp.max(zm, axis=1, keepdims=True)
    s = jnp.sum(jnp.where(valid, jnp.exp(z - m), 0.0), axis=1, keepdims=True)
    o_ref[...] = (z - m - jnp.log(s))[:, :NCLASS]


def kernel(x, src, tgt, Mtgt, W1, b1, W2, b2):
    pad = EP - E
    srcp = jnp.pad(src, (0, pad)).reshape(TOTCH, CH)
    tgtp = jnp.pad(tgt, (0, pad)).reshape(TOTCH, CH)
    mvp = jnp.pad(Mtgt, (0, pad)).reshape(TOTCH, CH)

    inv1 = _unpack_perm(NHID)
    inv2 = _unpack_perm(D2P)
    w1p = W1[:, inv1]
    w2p = jnp.pad(W2, ((0, 0), (0, D2P - NCLASS)))[:, inv2]
    b2p = jnp.pad(b2, (0, D2P - NCLASS))

    xp = jnp.pad(x, ((0, NP - N), (0, 0)))
    support1 = pl.pallas_call(
        _mm1_body,
        out_shape=jax.ShapeDtypeStruct((NP, NHID // 2), jnp.int32),
    )(xp, w1p)

    p0, p1 = _sc_agg_128(support1, srcp, tgtp, mvp)

    support2 = pl.pallas_call(
        _layer2_body,
        out_shape=jax.ShapeDtypeStruct((NP, D2P // 2), jnp.int32),
    )(p0, p1, b1, w2p)

    q0, q1 = _sc_agg_64(support2, srcp, tgtp, mvp)

    outp = pl.pallas_call(
        _final_body,
        out_shape=jax.ShapeDtypeStruct((NP, NCLASS), jnp.float32),
    )(q0, q1, b2p)

    return outp[:N]


# speedup vs baseline: 1.3874x; 1.1384x over previous
import functools

import jax
import jax.numpy as jnp
import numpy as np
from jax import lax
from jax.experimental import pallas as pl
from jax.experimental.pallas import tpu as pltpu
from jax.experimental.pallas import tpu_sc as plsc

N = 10000
NP = 10240
E = 320000
NFEAT = 128
NHID = 128
NCLASS = 40
D2P = 64

NC = 2
NS = 16
NW = NC * NS
CH = 64
NBG = 4
NBS = 2
GB = 32
KJ = 160
TOTCH = NW * KJ
EP = TOTCH * CH
RPT = NP // NS
ZR = CH


def _unpack_perm(D):
    perm = np.empty((D,), np.int64)
    for g in range(D // 32):
        for i in range(16):
            perm[32 * g + i] = 16 * g + i
            perm[32 * g + 16 + i] = D // 2 + 16 * g + i
    inv = np.empty_like(perm)
    inv[perm] = np.arange(D)
    return inv


def _make_sc_agg(D, k0):
    k1 = TOTCH // NS - k0
    mesh = plsc.VectorSubcoreMesh(core_axis_name="c", subcore_axis_name="s")

    @functools.partial(
        pl.kernel,
        out_type=[jax.ShapeDtypeStruct((NP, D), jnp.float32)] * NC,
        mesh=mesh,
        compiler_params=pltpu.CompilerParams(use_tc_tiling_on_sc=False),
        scratch_types=[
            pltpu.VMEM((GB, CH), jnp.int32),
            pltpu.VMEM((GB, CH), jnp.int32),
            pltpu.VMEM((GB, CH), jnp.float32),
            [pltpu.VMEM((CH, D // 2), jnp.int32)] * NBG,
            [pltpu.VMEM((CH, D), jnp.float32)] * NBS,
            pltpu.VMEM_SHARED((NP, D), jnp.float32),
            [pltpu.SemaphoreType.DMA] * NBG,
            [pltpu.SemaphoreType.DMA] * NBS,
        ],
    )
    def agg(support, srcm, tgtm, mvals, out0, out1, src_v, tgt_v, mv_v, rowsb, rowsf,
            acc, sg, ss):
        cid = lax.axis_index("c")
        sid = lax.axis_index("s")
        kc = jnp.where(cid == 0, k0, k1)
        crow = jnp.where(cid == 0, sid * k0, NS * k0 + sid * k1)

        zeros16 = jnp.zeros((16,), jnp.float32)

        def zrow(r, carry):
            for f in range(D // 16):
                rowsf[0][r, pl.ds(f * 16, 16)] = zeros16
            return carry

        lax.fori_loop(0, ZR, zrow, 0)

        base = sid * RPT

        def zcp(i, carry):
            pltpu.sync_copy(rowsf[0], acc.at[pl.ds(base + i * ZR, ZR), :])
            return carry

        lax.fori_loop(0, RPT // ZR, zcp, 0)
        plsc.subcore_barrier()

        dn = lax.GatherDimensionNumbers(
            offset_dims=(), collapsed_slice_dims=(0,), start_index_map=(0,))

        def g_start(j, b):
            pltpu.async_copy(support.at[src_v.at[j]], rowsb[b], sg[b])

        def g_wait(j, b):
            pltpu.make_async_copy(support.at[src_v.at[j]], rowsb[b], sg[b]).wait()

        def s_start(j, b):
            pltpu.async_copy(rowsf[b], acc.at[tgt_v.at[j]], ss[b], add=True)

        def s_wait(j, b):
            pltpu.make_async_copy(rowsf[b], acc.at[tgt_v.at[j]], ss[b]).wait()

        himask = jnp.full((16,), -65536, jnp.int32)

        def scale(j, bg, bs):
            def sgrp(g, c2):
                mv16 = mv_v[j, pl.ds(g * 16, 16)]
                for i in range(16):
                    m = lax.gather(
                        mv16, jnp.full((16, 1), i, jnp.int32), dn, (1,),
                        mode=lax.GatherScatterMode.PROMISE_IN_BOUNDS)
                    e = g * 16 + i
                    for g2 in range(D // 32):
                        w = rowsb[bg][e, pl.ds(g2 * 16, 16)]
                        lo = lax.bitcast_convert_type(w << 16, jnp.float32)
                        hi = lax.bitcast_convert_type(w & himask, jnp.float32)
                        rowsf[bs][e, pl.ds(g2 * 32, 16)] = lo * m
                        rowsf[bs][e, pl.ds(g2 * 32 + 16, 16)] = hi * m
                return c2

            lax.fori_loop(0, CH // 16, sgrp, 0)

        def group(gr, carry):
            row = crow + gr * GB
            pltpu.sync_copy(srcm.at[pl.ds(row, GB), :], src_v)
            pltpu.sync_copy(tgtm.at[pl.ds(row, GB), :], tgt_v)
            pltpu.sync_copy(mvals.at[pl.ds(row, GB), :], mv_v)

            for b in range(NBG - 1):
                g_start(b, b)

            def quad(q, c1):
                for b in range(NBG):
                    j = q * NBG + b
                    bs = b % NBS
                    g_wait(j, b)

                    if b >= NBS:
                        s_wait(j - NBS, bs)
                    else:
                        @pl.when(q > 0)
                        def _():
                            s_wait(j - NBS, bs)

                    scale(j, b, bs)
                    s_start(j, bs)
                    bp = (b + NBG - 1) % NBG

                    @pl.when(j + NBG - 1 < GB)
                    def _():
                        g_start(j + NBG - 1, bp)
                return c1

            lax.fori_loop(0, GB // NBG, quad, 0)
            s_wait(GB - NBS, (GB - NBS) % NBS)
            s_wait(GB - 1, (GB - 1) % NBS)
            return carry

        lax.fori_loop(0, kc // GB, group, 0)
        plsc.subcore_barrier()

        def wb0(i, carry):
            sl = pl.ds(base + i * ZR, ZR)
            pltpu.sync_copy(acc.at[sl, :], out0.at[sl, :])
            return carry

        def wb1(i, carry):
            sl = pl.ds(base + i * ZR, ZR)
            pltpu.sync_copy(acc.at[sl, :], out1.at[sl, :])
            return carry

        @pl.when(cid == 0)
        def _():
            lax.fori_loop(0, RPT // ZR, wb0, 0)

        @pl.when(cid == 1)
        def _():
            lax.fori_loop(0, RPT // ZR, wb1, 0)

    return agg


_sc_agg_128 = _make_sc_agg(NHID, 176)
_sc_agg_64 = _make_sc_agg(D2P, 176)


def _pack_halves(r):
    d = r.shape[1]
    a = r[:, : d // 2].astype(jnp.bfloat16)
    b = r[:, d // 2:].astype(jnp.bfloat16)
    au = lax.bitcast_convert_type(a, jnp.uint16).astype(jnp.uint32)
    bu = lax.bitcast_convert_type(b, jnp.uint16).astype(jnp.uint32)
    return lax.bitcast_convert_type(au | (bu << 16), jnp.int32)


def _mm1_body(x_ref, w_ref, o_ref):
    r = jnp.dot(x_ref[...], w_ref[...], preferred_element_type=jnp.float32)
    o_ref[...] = _pack_halves(r)


def _layer2_body(p0_ref, p1_ref, b1_ref, w2_ref, o_ref):
    h = jax.nn.relu(p0_ref[...] + p1_ref[...] + b1_ref[...])
    r = jnp.dot(h, w2_ref[...], preferred_element_type=jnp.float32)
    o_ref[...] = _pack_halves(r)


def _final_body(q0_ref, q1_ref, b2_ref, o_ref):
    z = q0_ref[...] + q1_ref[...] + b2_ref[...]
    col = lax.broadcasted_iota(jnp.int32, (NP, D2P), 1)
    valid = col < NCLASS
    zm = jnp.where(valid, z, -jnp.inf)
    m = jnp.max(zm, axis=1, keepdims=True)
    s = jnp.sum(jnp.where(valid, jnp.exp(z - m), 0.0), axis=1, keepdims=True)
    o_ref[...] = (z - m - jnp.log(s))[:, :NCLASS]


def kernel(x, src, tgt, Mtgt, W1, b1, W2, b2):
    pad = EP - E
    srcp = jnp.pad(src, (0, pad)).reshape(TOTCH, CH)
    tgtp = jnp.pad(tgt, (0, pad)).reshape(TOTCH, CH)
    mvp = jnp.pad(Mtgt, (0, pad)).reshape(TOTCH, CH)

    inv1 = _unpack_perm(NHID)
    inv2 = _unpack_perm(D2P)
    w1p = W1[:, inv1]
    w2p = jnp.pad(W2, ((0, 0), (0, D2P - NCLASS)))[:, inv2]
    b2p = jnp.pad(b2, (0, D2P - NCLASS))

    xp = jnp.pad(x, ((0, NP - N), (0, 0)))
    support1 = pl.pallas_call(
        _mm1_body,
        out_shape=jax.ShapeDtypeStruct((NP, NHID // 2), jnp.int32),
    )(xp, w1p)

    p0, p1 = _sc_agg_128(support1, srcp, tgtp, mvp)

    support2 = pl.pallas_call(
        _layer2_body,
        out_shape=jax.ShapeDtypeStruct((NP, D2P // 2), jnp.int32),
    )(p0, p1, b1, w2p)

    q0, q1 = _sc_agg_64(support2, srcp, tgtp, mvp)

    outp = pl.pallas_call(
        _final_body,
        out_shape=jax.ShapeDtypeStruct((NP, NCLASS), jnp.float32),
    )(q0, q1, b2p)

    return outp[:N]
